# Initial kernel scaffold; baseline (speedup 1.0000x reference)
import jax
import jax.numpy as jnp
from jax.experimental import pallas as pl


def kernel(x, edge_index, Wq, bq, Wk, bk, Wv, bv, Wo, bo, gamma1, beta1, gamma2, beta2):
    raise NotImplementedError("write your pallas kernel here")


# speedup vs baseline: 12.4576x; 12.4576x over previous
import functools

import jax
import jax.numpy as jnp
from jax import lax
from jax.experimental import pallas as pl
from jax.experimental.pallas import tpu as pltpu
from jax.experimental.pallas import tpu_sc as plsc

N = 10000
E = 320000
D = 128
H = 8
DH = 16

NC = 2
NS = 16
NW = NC * NS
EPW = E // NW
CH = 80
NCHUNK = EPW // CH
NPAD = 10240
RPS = NPAD // NS
RZB = 128


def _qkv_body(x_ref, w_ref, b_ref, q_ref, k_ref, v_ref):
    acc = jnp.dot(x_ref[...], w_ref[...],
                  preferred_element_type=jnp.float32) + b_ref[...]
    q_ref[...] = acc[:, :D]
    k_ref[...] = acc[:, D:2 * D]
    v_ref[...] = acc[:, 2 * D:]


def _qkv(x, w, b):
    blk = 1000
    return pl.pallas_call(
        _qkv_body,
        grid=(N // blk,),
        in_specs=[
            pl.BlockSpec((blk, D), lambda i: (i, 0)),
            pl.BlockSpec((D, 3 * D), lambda i: (0, 0)),
            pl.BlockSpec((1, 3 * D), lambda i: (0, 0)),
        ],
        out_specs=[pl.BlockSpec((blk, D), lambda i: (i, 0))] * 3,
        out_shape=[jax.ShapeDtypeStruct((N, D), jnp.float32)] * 3,
    )(x, w, b)


def _edge_body(q_hbm, k_hbm, v_hbm, src_hbm, dst_hbm, out_hbm,
               idx_src, idx_dst, krows, qrows, vrows, msg, wv_sh, sem):
    c = lax.axis_index("c")
    s = lax.axis_index("s")
    w = c * NS + s

    zero16 = jnp.zeros((16,), jnp.float32)

    def _zrow(r, carry):
        for g in range(D // 16):
            msg[r, pl.ds(g * 16, 16)] = zero16
        return carry

    lax.fori_loop(0, CH, _zrow, 0)
    for j in range(RPS // CH):
        pltpu.sync_copy(msg, wv_sh.at[pl.ds(s * RPS + j * CH, CH)])
    plsc.subcore_barrier()

    iota16 = lax.iota(jnp.int32, 16)

    def _chunk(i, carry):
        base = w * EPW + i * CH
        pltpu.sync_copy(src_hbm.at[pl.ds(base, CH)], idx_src)
        pltpu.sync_copy(dst_hbm.at[pl.ds(base, CH)], idx_dst)
        cp_k = pltpu.async_copy(k_hbm.at[idx_src], krows, sem)
        cp_q = pltpu.async_copy(q_hbm.at[idx_dst], qrows, sem)
        cp_v = pltpu.async_copy(v_hbm.at[idx_src], vrows, sem)
        cp_k.wait()
        cp_q.wait()
        cp_v.wait()

        def _grp(g, carry2):
            rows = g * 16 + iota16
            for h in range(H):
                acc = jnp.zeros((16,), jnp.float32)
                for dh in range(DH):
                    col = jnp.full((16,), h * DH + dh, jnp.int32)
                    kv = plsc.load_gather(krows, [rows, col])
                    qv = plsc.load_gather(qrows, [rows, col])
                    acc = acc + kv * qv
                sc = jnp.exp(jnp.clip(acc * 0.25, -5.0, 5.0))
                for dh in range(DH):
                    col = jnp.full((16,), h * DH + dh, jnp.int32)
                    vv = plsc.load_gather(vrows, [rows, col])
                    plsc.store_scatter(msg, [rows, col], vv * sc)
            return carry2

        lax.fori_loop(0, CH // 16, _grp, 0)
        pltpu.sync_copy(msg, wv_sh.at[idx_dst], add=True)
        return carry

    lax.fori_loop(0, NCHUNK, _chunk, 0)

    plsc.subcore_barrier()
    for j in range(RPS // RZB):
        r0 = s * RPS + j * RZB
        pltpu.sync_copy(wv_sh.at[pl.ds(r0, RZB)], out_hbm.at[c, pl.ds(r0, RZB)])


_edge = functools.partial(
    pl.kernel,
    out_type=jax.ShapeDtypeStruct((NC, NPAD, D), jnp.float32),
    mesh=plsc.VectorSubcoreMesh(core_axis_name="c", subcore_axis_name="s",
                                num_cores=NC, num_subcores=NS),
    compiler_params=pltpu.CompilerParams(needs_layout_passes=False),
    scratch_types=[
        pltpu.VMEM((CH,), jnp.int32),
        pltpu.VMEM((CH,), jnp.int32),
        pltpu.VMEM((CH, D), jnp.float32),
        pltpu.VMEM((CH, D), jnp.float32),
        pltpu.VMEM((CH, D), jnp.float32),
        pltpu.VMEM((CH, D), jnp.float32),
        pltpu.VMEM_SHARED((NPAD, D), jnp.float32),
        pltpu.SemaphoreType.DMA,
    ],
)(_edge_body)


def _tail_body(x_ref, wv_ref, g1_ref, b1_ref, g2_ref, b2_ref, wo_ref, bo_ref,
               out_ref):
    h = x_ref[...] + jnp.sum(wv_ref[...], axis=0)
    mu = jnp.mean(h, axis=-1, keepdims=True)
    var = jnp.mean((h - mu) ** 2, axis=-1, keepdims=True)
    hn = (h - mu) / jnp.sqrt(var + 1e-5) * g1_ref[...] + b1_ref[...]
    mu2 = jnp.mean(hn, axis=-1, keepdims=True)
    var2 = jnp.mean((hn - mu2) ** 2, axis=-1, keepdims=True)
    t = (hn - mu2) / jnp.sqrt(var2 + 1e-5) * g2_ref[...] + b2_ref[...]
    ff = jnp.dot(t, wo_ref[...], preferred_element_type=jnp.float32) + bo_ref[...]
    out_ref[...] = hn + jnp.maximum(ff, 0.0)


def _tail(x, wv2, g1, b1, g2, b2, wo, bo):
    blk = 1000
    return pl.pallas_call(
        _tail_body,
        grid=(N // blk,),
        in_specs=[
            pl.BlockSpec((blk, D), lambda i: (i, 0)),
            pl.BlockSpec((NC, blk, D), lambda i: (0, i, 0)),
            pl.BlockSpec((1, D), lambda i: (0, 0)),
            pl.BlockSpec((1, D), lambda i: (0, 0)),
            pl.BlockSpec((1, D), lambda i: (0, 0)),
            pl.BlockSpec((1, D), lambda i: (0, 0)),
            pl.BlockSpec((D, D), lambda i: (0, 0)),
            pl.BlockSpec((1, D), lambda i: (0, 0)),
        ],
        out_specs=pl.BlockSpec((blk, D), lambda i: (i, 0)),
        out_shape=jax.ShapeDtypeStruct((N, D), jnp.float32),
    )(x, wv2, g1, b1, g2, b2, wo, bo)


def kernel(x, edge_index, Wq, bq, Wk, bk, Wv, bv, Wo, bo,
           gamma1, beta1, gamma2, beta2):
    src = edge_index[0].astype(jnp.int32)
    dst = edge_index[1].astype(jnp.int32)
    w = jnp.concatenate([Wq, Wk, Wv], axis=1)
    b = jnp.concatenate([bq, bk, bv]).reshape(1, 3 * D)
    q, k, v = _qkv(x, w, b)
    wv2 = _edge(q, k, v, src, dst)
    return _tail(x, wv2,
                 gamma1.reshape(1, D), beta1.reshape(1, D),
                 gamma2.reshape(1, D), beta2.reshape(1, D),
                 Wo, bo.reshape(1, D))


# speedup vs baseline: 13.1949x; 1.0592x over previous
import functools

import jax
import jax.numpy as jnp
from jax import lax
from jax.experimental import pallas as pl
from jax.experimental.pallas import tpu as pltpu
from jax.experimental.pallas import tpu_sc as plsc

N = 10000
E = 320000
D = 128
H = 8
DH = 16

NC = 2
NS = 16
NW = NC * NS
EPW = E // NW
CH = 32
SBC = 8
SB_E = SBC * CH
NCHUNKS = 312
NSB = NCHUNKS // SBC
TAIL = EPW - NCHUNKS * CH
NPAD = 10240
RPS = NPAD // NS
ZROWS = 40
RZB = 128


def _qkv_body(x_ref, w_ref, b_ref, q_ref, kv_ref):
    acc = jnp.dot(x_ref[...], w_ref[...],
                  preferred_element_type=jnp.float32) + b_ref[...]
    q_ref[...] = acc[:, :D]
    kv_ref[...] = acc[:, D:]


def _qkv(x, w, b):
    blk = 1000
    return pl.pallas_call(
        _qkv_body,
        grid=(N // blk,),
        in_specs=[
            pl.BlockSpec((blk, D), lambda i: (i, 0)),
            pl.BlockSpec((D, 3 * D), lambda i: (0, 0)),
            pl.BlockSpec((1, 3 * D), lambda i: (0, 0)),
        ],
        out_specs=[pl.BlockSpec((blk, D), lambda i: (i, 0)),
                   pl.BlockSpec((blk, 2 * D), lambda i: (i, 0))],
        out_shape=[jax.ShapeDtypeStruct((N, D), jnp.float32),
                   jax.ShapeDtypeStruct((N, 2 * D), jnp.float32)],
    )(x, w, b)


def _edge_body(q_hbm, kv_hbm, src_hbm, dst_hbm, out_hbm,
               is0, is1, id0, id1, td, kvr, qr, msg, wv_sh, sem_i, sem_g):
    c = lax.axis_index("c")
    s = lax.axis_index("s")
    w = c * NS + s
    ebase = w * EPW

    zero16 = jnp.zeros((16,), jnp.float32)
    iota16 = lax.iota(jnp.int32, 16)

    def _zrow(r, carry):
        for g in range(D // 16):
            msg[r, pl.ds(g * 16, 16)] = zero16
        return carry

    lax.fori_loop(0, ZROWS, _zrow, 0)
    for j in range(RPS // ZROWS):
        pltpu.sync_copy(msg.at[pl.ds(0, ZROWS)],
                        wv_sh.at[pl.ds(s * RPS + j * ZROWS, ZROWS)])
    plsc.subcore_barrier()

    def _score_msg(kvr_ref, qr_ref, rows):
        def _head(h, carry):
            base = h * DH
            acc = zero16
            for dh in range(DH):
                col = jnp.full((16,), base + dh, jnp.int32)
                acc = acc + (plsc.load_gather(kvr_ref, [rows, col]) *
                             plsc.load_gather(qr_ref, [rows, col]))
            scv = jnp.exp(jnp.clip(acc * 0.25, -5.0, 5.0))
            for dh in range(DH):
                colm = jnp.full((16,), base + dh, jnp.int32)
                vv = plsc.load_gather(kvr_ref, [rows, colm + D])
                plsc.store_scatter(msg, [rows, colm], vv * scv)
            return carry

        lax.fori_loop(0, H, _head, 0)

    pltpu.sync_copy(src_hbm.at[pl.ds(ebase, CH)], is0)
    pltpu.sync_copy(dst_hbm.at[pl.ds(ebase, CH)], id0)
    pltpu.async_copy(src_hbm.at[pl.ds(ebase + CH, CH)], is1, sem_i.at[1])
    pltpu.async_copy(dst_hbm.at[pl.ds(ebase + CH, CH)], id1, sem_i.at[1])
    pltpu.async_copy(kv_hbm.at[is0], kvr.at[0], sem_g.at[0])
    pltpu.async_copy(q_hbm.at[id0], qr.at[0], sem_g.at[0])

    def _chunk(i, carry):
        b = lax.rem(i, 2)
        more = i < NCHUNKS - 1

        @pl.when(jnp.logical_and(more, b == 0))
        def _():
            off = pl.ds(ebase + (i + 1) * CH, CH)
            pltpu.make_async_copy(src_hbm.at[off], is1, sem_i.at[1]).wait()
            pltpu.make_async_copy(dst_hbm.at[off], id1, sem_i.at[1]).wait()
            pltpu.async_copy(kv_hbm.at[is1], kvr.at[1], sem_g.at[1])
            pltpu.async_copy(q_hbm.at[id1], qr.at[1], sem_g.at[1])

        @pl.when(jnp.logical_and(more, b == 1))
        def _():
            off = pl.ds(ebase + (i + 1) * CH, CH)
            pltpu.make_async_copy(src_hbm.at[off], is0, sem_i.at[0]).wait()
            pltpu.make_async_copy(dst_hbm.at[off], id0, sem_i.at[0]).wait()
            pltpu.async_copy(kv_hbm.at[is0], kvr.at[0], sem_g.at[0])
            pltpu.async_copy(q_hbm.at[id0], qr.at[0], sem_g.at[0])

        pltpu.make_async_copy(kv_hbm.at[is0], kvr.at[b], sem_g.at[b]).wait()
        pltpu.make_async_copy(q_hbm.at[id0], qr.at[b], sem_g.at[b]).wait()

        def _grp(g, carry2):
            _score_msg(kvr.at[b], qr.at[b], g * 16 + iota16)
            return carry2

        lax.fori_loop(0, CH // 16, _grp, 0)

        @pl.when(b == 0)
        def _():
            pltpu.sync_copy(msg, wv_sh.at[id0], add=True)

        @pl.when(b == 1)
        def _():
            pltpu.sync_copy(msg, wv_sh.at[id1], add=True)

        @pl.when(jnp.logical_and(i < NCHUNKS - 2, b == 0))
        def _():
            off = pl.ds(ebase + (i + 2) * CH, CH)
            pltpu.async_copy(src_hbm.at[off], is0, sem_i.at[0])
            pltpu.async_copy(dst_hbm.at[off], id0, sem_i.at[0])

        @pl.when(jnp.logical_and(i < NCHUNKS - 2, b == 1))
        def _():
            off = pl.ds(ebase + (i + 2) * CH, CH)
            pltpu.async_copy(src_hbm.at[off], is1, sem_i.at[1])
            pltpu.async_copy(dst_hbm.at[off], id1, sem_i.at[1])
        return carry

    lax.fori_loop(0, NCHUNKS, _chunk, 0)

    tb = ebase + NCHUNKS * CH
    pltpu.sync_copy(src_hbm.at[pl.ds(tb, TAIL)], is0.at[pl.ds(0, TAIL)])
    pltpu.sync_copy(dst_hbm.at[pl.ds(tb, TAIL)], td)
    cp1 = pltpu.async_copy(kv_hbm.at[is0.at[pl.ds(0, TAIL)]],
                           kvr.at[0, pl.ds(0, TAIL)], sem_g.at[0])
    cp2 = pltpu.async_copy(q_hbm.at[td], qr.at[0, pl.ds(0, TAIL)],
                           sem_g.at[0])
    cp1.wait()
    cp2.wait()
    _score_msg(kvr.at[0], qr.at[0], iota16)
    pltpu.sync_copy(msg.at[pl.ds(0, TAIL)], wv_sh.at[td], add=True)

    plsc.subcore_barrier()
    for j in range(RPS // RZB):
        r0 = s * RPS + j * RZB
        pltpu.sync_copy(wv_sh.at[pl.ds(r0, RZB)], out_hbm.at[c, pl.ds(r0, RZB)])


_edge = functools.partial(
    pl.kernel,
    out_type=jax.ShapeDtypeStruct((NC, NPAD, D), jnp.float32),
    mesh=plsc.VectorSubcoreMesh(core_axis_name="c", subcore_axis_name="s",
                                num_cores=NC, num_subcores=NS),
    compiler_params=pltpu.CompilerParams(needs_layout_passes=False),
    scratch_types=[
        pltpu.VMEM((CH,), jnp.int32),
        pltpu.VMEM((CH,), jnp.int32),
        pltpu.VMEM((CH,), jnp.int32),
        pltpu.VMEM((CH,), jnp.int32),
        pltpu.VMEM((TAIL,), jnp.int32),
        pltpu.VMEM((2, CH, 2 * D), jnp.float32),
        pltpu.VMEM((2, CH, D), jnp.float32),
        pltpu.VMEM((CH, D), jnp.float32),
        pltpu.VMEM_SHARED((NPAD, D), jnp.float32),
        pltpu.SemaphoreType.DMA((2,)),
        pltpu.SemaphoreType.DMA((2,)),
    ],
)(_edge_body)


def _tail_body(x_ref, wv_ref, g1_ref, b1_ref, g2_ref, b2_ref, wo_ref, bo_ref,
               out_ref):
    h = x_ref[...] + jnp.sum(wv_ref[...], axis=0)
    mu = jnp.mean(h, axis=-1, keepdims=True)
    var = jnp.mean((h - mu) ** 2, axis=-1, keepdims=True)
    hn = (h - mu) / jnp.sqrt(var + 1e-5) * g1_ref[...] + b1_ref[...]
    mu2 = jnp.mean(hn, axis=-1, keepdims=True)
    var2 = jnp.mean((hn - mu2) ** 2, axis=-1, keepdims=True)
    t = (hn - mu2) / jnp.sqrt(var2 + 1e-5) * g2_ref[...] + b2_ref[...]
    ff = jnp.dot(t, wo_ref[...], preferred_element_type=jnp.float32) + bo_ref[...]
    out_ref[...] = hn + jnp.maximum(ff, 0.0)


def _tail(x, wv2, g1, b1, g2, b2, wo, bo):
    blk = 1000
    return pl.pallas_call(
        _tail_body,
        grid=(N // blk,),
        in_specs=[
            pl.BlockSpec((blk, D), lambda i: (i, 0)),
            pl.BlockSpec((NC, blk, D), lambda i: (0, i, 0)),
            pl.BlockSpec((1, D), lambda i: (0, 0)),
            pl.BlockSpec((1, D), lambda i: (0, 0)),
            pl.BlockSpec((1, D), lambda i: (0, 0)),
            pl.BlockSpec((1, D), lambda i: (0, 0)),
            pl.BlockSpec((D, D), lambda i: (0, 0)),
            pl.BlockSpec((1, D), lambda i: (0, 0)),
        ],
        out_specs=pl.BlockSpec((blk, D), lambda i: (i, 0)),
        out_shape=jax.ShapeDtypeStruct((N, D), jnp.float32),
    )(x, wv2, g1, b1, g2, b2, wo, bo)


def kernel(x, edge_index, Wq, bq, Wk, bk, Wv, bv, Wo, bo,
           gamma1, beta1, gamma2, beta2):
    src = edge_index[0].astype(jnp.int32)
    dst = edge_index[1].astype(jnp.int32)
    w = jnp.concatenate([Wq, Wk, Wv], axis=1)
    b = jnp.concatenate([bq, bk, bv]).reshape(1, 3 * D)
    q, kv = _qkv(x, w, b)
    wv2 = _edge(q, kv, src, dst)
    return _tail(x, wv2,
                 gamma1.reshape(1, D), beta1.reshape(1, D),
                 gamma2.reshape(1, D), beta2.reshape(1, D),
                 Wo, bo.reshape(1, D))
